# baseline (speedup 1.0000x reference)
import jax
import jax.numpy as jnp
from jax import lax
from jax.experimental import pallas as pl
from jax.experimental.pallas import tpu as pltpu
from jax.experimental.pallas import tpu_sc as plsc

_N, _L, _D = 10000, 128, 128
_NBUF = 4
_G = 16
_DEPTH = 8


def _make_body(B, NW, NC):
    bpw = B // NW
    nchunks = bpw
    nvar = _L // _G

    def body(idx_hbm, idxp_hbm, table_hbm, len_hbm, out_hbm, mask_hbm,
             idx_v, idxp_v, lentab_v, lens_v, nv_v, buf0, buf1, buf2, buf3,
             zbuf, maskfull, gs0, gs1, gs2, gs3, ss0, ss1, ss2, ss3, zsem):
        bufs = (buf0, buf1, buf2, buf3)
        gsems = (gs0, gs1, gs2, gs3)
        ssems = (ss0, ss1, ss2, ss3)
        wid = lax.axis_index("s") * NC + lax.axis_index("c")
        base = wid * bpw
        pltpu.sync_copy(idx_hbm.at[pl.ds(base, bpw)], idx_v)
        pltpu.sync_copy(idxp_hbm.at[wid], idxp_v)
        pltpu.sync_copy(len_hbm, lentab_v)
        iota = lax.iota(jnp.int32, 16)
        zero16 = jnp.zeros((16,), jnp.float32)

        def zb_step(r, c):
            for j in range(_D // 16):
                zbuf[0, r, pl.ds(j * 16, 16)] = zero16
            return c
        lax.fori_loop(0, _L - _G, zb_step, 0)

        def lens_step(g, c):
            iv = idx_v[pl.ds(g * 16, 16)]
            lv = plsc.load_gather(lentab_v, [iv])
            lens_v[pl.ds(g * 16, 16)] = lv
            nv_v[pl.ds(g * 16, 16)] = lax.shift_right_logical(
                lv + (_G - 1), 4)
            return c
        lax.fori_loop(0, bpw // 16, lens_step, 0)

        def extract(ref, b):
            vs = (b // 16) * 16
            lv = ref[pl.ds(vs, 16)]
            return jnp.sum(jnp.where(iota == b - vs, lv, 0))

        def gather_start(c, b):
            gidx = idxp_v.at[pl.ds(c * 8, 1)]
            nv = extract(nv_v, c)
            for k in range(1, nvar + 1):
                @pl.when(nv == k)
                def _():
                    pltpu.make_async_copy(
                        table_hbm.at[gidx, pl.ds(0, k * _G)],
                        bufs[b].at[:, pl.ds(0, k * _G)],
                        gsems[b]).start()

        def gather_wait(c, b):
            gidx = idxp_v.at[pl.ds(c * 8, 1)]
            nv = extract(nv_v, c)
            for k in range(1, nvar + 1):
                @pl.when(nv == k)
                def _():
                    pltpu.make_async_copy(
                        table_hbm.at[gidx, pl.ds(0, k * _G)],
                        bufs[b].at[:, pl.ds(0, k * _G)],
                        gsems[b]).wait()

        def data_scatter(c, b, start):
            nv = extract(nv_v, c)
            for k in range(1, nvar + 1):
                @pl.when(nv == k)
                def _():
                    cp = pltpu.make_async_copy(
                        bufs[b].at[:, pl.ds(0, k * _G)],
                        out_hbm.at[pl.ds(base + c, 1), pl.ds(0, k * _G)],
                        ssems[b])
                    cp.start() if start else cp.wait()

        def zero_scatter(c, start):
            nv = extract(nv_v, c)
            for k in range(1, nvar):
                @pl.when(nv == k)
                def _():
                    nz = (nvar - k) * _G
                    cp = pltpu.make_async_copy(
                        zbuf.at[:, pl.ds(0, nz)],
                        out_hbm.at[pl.ds(base + c, 1), pl.ds(k * _G, nz)],
                        zsem)
                    cp.start() if start else cp.wait()

        for b in range(_NBUF):
            gather_start(b, b)

        def mask_step(b, c):
            len_s = extract(lens_v, b)
            for j in range(_L // 16):
                maskfull[b, pl.ds(j * 16, 16)] = jnp.where(
                    j * 16 + iota < len_s, jnp.int32(1), jnp.int32(0))
            return c
        lax.fori_loop(0, bpw, mask_step, 0)

        def chunk_work(c, b):
            nv = extract(nv_v, c)
            len_s = extract(lens_v, c)
            gidx = idxp_v.at[pl.ds(c * 8, 1)]

            def zrow(r, cc):
                for j in range(_D // 16):
                    bufs[b][0, r, pl.ds(j * 16, 16)] = zero16
                return cc

            for k in range(1, nvar + 1):
                @pl.when(nv == k)
                def _():
                    pltpu.make_async_copy(
                        table_hbm.at[gidx, pl.ds(0, k * _G)],
                        bufs[b].at[:, pl.ds(0, k * _G)],
                        gsems[b]).wait()
                    lax.fori_loop(len_s, k * _G, zrow, 0)
                    pltpu.make_async_copy(
                        bufs[b].at[:, pl.ds(0, k * _G)],
                        out_hbm.at[pl.ds(base + c, 1), pl.ds(0, k * _G)],
                        ssems[b]).start()
                    if k < nvar:
                        nz = (nvar - k) * _G
                        pltpu.make_async_copy(
                            zbuf.at[:, pl.ds(0, nz)],
                            out_hbm.at[pl.ds(base + c, 1),
                                       pl.ds(k * _G, nz)],
                            zsem).start()

            @pl.when(c + _NBUF < nchunks)
            def _():
                for k in range(1, nvar + 1):
                    @pl.when(nv == k)
                    def _():
                        pltpu.make_async_copy(
                            bufs[b].at[:, pl.ds(0, k * _G)],
                            out_hbm.at[pl.ds(base + c, 1),
                                       pl.ds(0, k * _G)],
                            ssems[b]).wait()
                gather_start(c + _NBUF, b)

            @pl.when(c >= _DEPTH)
            def _():
                zero_scatter(c - _DEPTH, False)

        def step(i, carry):
            for b in range(_NBUF):
                chunk_work(i * _NBUF + b, b)
            return carry
        lax.fori_loop(0, nchunks // _NBUF, step, 0)

        for b in range(_NBUF):
            data_scatter(nchunks - _NBUF + b, b, False)

        def drain(c, carry):
            zero_scatter(c, False)
            return carry
        lax.fori_loop(nchunks - _DEPTH, nchunks, drain, 0)

        pltpu.sync_copy(maskfull, mask_hbm.at[pl.ds(base, bpw)])

    return body


def kernel(indices, table, lengths):
    B = indices.shape[0]
    info = plsc.get_sparse_core_info()
    NC, NS = info.num_cores, info.num_subcores
    NW = NC * NS
    bpw = B // NW
    idx_pad = jnp.zeros((B, 8), jnp.int32)
    idx_pad = idx_pad.at[:, 0].set(indices)
    idx_pad = idx_pad.reshape(NW, bpw * 8)
    run = pl.kernel(
        _make_body(B, NW, NC),
        out_type=(
            jax.ShapeDtypeStruct((B, _L, _D), jnp.float32),
            jax.ShapeDtypeStruct((B, _L), jnp.int32),
        ),
        mesh=plsc.VectorSubcoreMesh(core_axis_name="c", subcore_axis_name="s"),
        compiler_params=pltpu.CompilerParams(needs_layout_passes=False),
        scratch_types=[
            pltpu.VMEM((B // NW,), jnp.int32),
            pltpu.VMEM((bpw * 8,), jnp.int32),
            pltpu.VMEM((_N,), jnp.int32),
            pltpu.VMEM((B // NW,), jnp.int32),
            pltpu.VMEM((B // NW,), jnp.int32),
            pltpu.VMEM((1, _L, _D), jnp.float32),
            pltpu.VMEM((1, _L, _D), jnp.float32),
            pltpu.VMEM((1, _L, _D), jnp.float32),
            pltpu.VMEM((1, _L, _D), jnp.float32),
            pltpu.VMEM((1, _L - _G, _D), jnp.float32),
            pltpu.VMEM((B // NW, _L), jnp.int32),
            pltpu.SemaphoreType.DMA,
            pltpu.SemaphoreType.DMA,
            pltpu.SemaphoreType.DMA,
            pltpu.SemaphoreType.DMA,
            pltpu.SemaphoreType.DMA,
            pltpu.SemaphoreType.DMA,
            pltpu.SemaphoreType.DMA,
            pltpu.SemaphoreType.DMA,
            pltpu.SemaphoreType.DMA,
        ],
    )
    padded, mask_i = run(indices, idx_pad, table, lengths)
    return padded, mask_i.astype(jnp.bool_)


# speedup vs baseline: 1.0107x; 1.0107x over previous
import jax
import jax.numpy as jnp
from jax import lax
from jax.experimental import pallas as pl
from jax.experimental.pallas import tpu as pltpu
from jax.experimental.pallas import tpu_sc as plsc

_N, _L, _D = 10000, 128, 128
_NBUF = 4
_G = 16
_DEPTH = 8


def _make_body(B, NW, NC):
    bpw = B // NW
    nchunks = bpw
    nvar = _L // _G

    def body(idx_hbm, idxp_hbm, table_hbm, len_hbm, out_hbm, mask_hbm,
             idx_v, idxp_v, lentab_v, lens_v, nv_v, buf0, buf1, buf2, buf3,
             zbuf, maskfull, gs0, gs1, gs2, gs3, ss0, ss1, ss2, ss3, zsem):
        bufs = (buf0, buf1, buf2, buf3)
        gsems = (gs0, gs1, gs2, gs3)
        ssems = (ss0, ss1, ss2, ss3)
        wid = lax.axis_index("s") * NC + lax.axis_index("c")
        base = wid * bpw
        pltpu.sync_copy(idx_hbm.at[pl.ds(base, bpw)], idx_v)
        pltpu.sync_copy(idxp_hbm.at[wid], idxp_v)
        pltpu.sync_copy(len_hbm, lentab_v)
        iota = lax.iota(jnp.int32, 16)
        zero16 = jnp.zeros((16,), jnp.float32)

        def zb_step(r, c):
            for j in range(_D // 16):
                zbuf[0, r, pl.ds(j * 16, 16)] = zero16
            return c
        lax.fori_loop(0, _L - _G, zb_step, 0)

        def lens_step(g, c):
            iv = idx_v[pl.ds(g * 16, 16)]
            lv = plsc.load_gather(lentab_v, [iv])
            lens_v[pl.ds(g * 16, 16)] = lv
            nv_v[pl.ds(g * 16, 16)] = lax.shift_right_logical(
                lv + (_G - 1), 4)
            return c
        lax.fori_loop(0, bpw // 16, lens_step, 0)

        def extract(ref, b):
            vs = (b // 16) * 16
            lv = ref[pl.ds(vs, 16)]
            return jnp.sum(jnp.where(iota == b - vs, lv, 0))

        def gather_start(c, b):
            gidx = idxp_v.at[pl.ds(c * 8, 1)]
            nv = extract(nv_v, c)
            for k in range(1, nvar + 1):
                @pl.when(nv == k)
                def _():
                    pltpu.make_async_copy(
                        table_hbm.at[gidx, pl.ds(0, k * _G)],
                        bufs[b].at[:, pl.ds(0, k * _G)],
                        gsems[b]).start()

        def gather_wait(c, b):
            gidx = idxp_v.at[pl.ds(c * 8, 1)]
            nv = extract(nv_v, c)
            for k in range(1, nvar + 1):
                @pl.when(nv == k)
                def _():
                    pltpu.make_async_copy(
                        table_hbm.at[gidx, pl.ds(0, k * _G)],
                        bufs[b].at[:, pl.ds(0, k * _G)],
                        gsems[b]).wait()

        def data_scatter(c, b, start):
            nv = extract(nv_v, c)
            for k in range(1, nvar + 1):
                @pl.when(nv == k)
                def _():
                    cp = pltpu.make_async_copy(
                        bufs[b].at[:, pl.ds(0, k * _G)],
                        out_hbm.at[pl.ds(base + c, 1), pl.ds(0, k * _G)],
                        ssems[b])
                    cp.start() if start else cp.wait()

        def zero_scatter(c, start):
            nv = extract(nv_v, c)
            for k in range(1, nvar):
                @pl.when(nv == k)
                def _():
                    nz = (nvar - k) * _G
                    cp = pltpu.make_async_copy(
                        zbuf.at[:, pl.ds(0, nz)],
                        out_hbm.at[pl.ds(base + c, 1), pl.ds(k * _G, nz)],
                        zsem)
                    cp.start() if start else cp.wait()

        for b in range(_NBUF):
            gather_start(b, b)

        def mask_step(b, c):
            len_s = extract(lens_v, b)
            for j in range(_L // 16):
                maskfull[b, pl.ds(j * 16, 16)] = jnp.where(
                    j * 16 + iota < len_s, jnp.int32(1), jnp.int32(0))
            return c
        lax.fori_loop(0, bpw, mask_step, 0)

        def chunk_work(c, b):
            gather_wait(c, b)
            len_s = extract(lens_v, c)
            nv16 = extract(nv_v, c) * _G

            def zrow(r, cc):
                for j in range(_D // 16):
                    bufs[b][0, r, pl.ds(j * 16, 16)] = zero16
                return cc
            lax.fori_loop(len_s, nv16, zrow, 0)

            data_scatter(c, b, True)
            zero_scatter(c, True)

            @pl.when(c + _NBUF < nchunks)
            def _():
                data_scatter(c, b, False)
                gather_start(c + _NBUF, b)

            @pl.when(c >= _DEPTH)
            def _():
                zero_scatter(c - _DEPTH, False)

        def step(i, carry):
            for b in range(_NBUF):
                chunk_work(i * _NBUF + b, b)
            return carry
        lax.fori_loop(0, nchunks // _NBUF, step, 0)

        for b in range(_NBUF):
            data_scatter(nchunks - _NBUF + b, b, False)

        def drain(c, carry):
            zero_scatter(c, False)
            return carry
        lax.fori_loop(nchunks - _DEPTH, nchunks, drain, 0)

        pltpu.sync_copy(maskfull, mask_hbm.at[pl.ds(base, bpw)])

    return body


def kernel(indices, table, lengths):
    B = indices.shape[0]
    info = plsc.get_sparse_core_info()
    NC, NS = info.num_cores, info.num_subcores
    NW = NC * NS
    bpw = B // NW
    idx_pad = jnp.zeros((B, 8), jnp.int32)
    idx_pad = idx_pad.at[:, 0].set(indices)
    idx_pad = idx_pad.reshape(NW, bpw * 8)
    run = pl.kernel(
        _make_body(B, NW, NC),
        out_type=(
            jax.ShapeDtypeStruct((B, _L, _D), jnp.float32),
            jax.ShapeDtypeStruct((B, _L), jnp.int32),
        ),
        mesh=plsc.VectorSubcoreMesh(core_axis_name="c", subcore_axis_name="s"),
        compiler_params=pltpu.CompilerParams(needs_layout_passes=False),
        scratch_types=[
            pltpu.VMEM((B // NW,), jnp.int32),
            pltpu.VMEM((bpw * 8,), jnp.int32),
            pltpu.VMEM((_N,), jnp.int32),
            pltpu.VMEM((B // NW,), jnp.int32),
            pltpu.VMEM((B // NW,), jnp.int32),
            pltpu.VMEM((1, _L, _D), jnp.float32),
            pltpu.VMEM((1, _L, _D), jnp.float32),
            pltpu.VMEM((1, _L, _D), jnp.float32),
            pltpu.VMEM((1, _L, _D), jnp.float32),
            pltpu.VMEM((1, _L - _G, _D), jnp.float32),
            pltpu.VMEM((B // NW, _L), jnp.int32),
            pltpu.SemaphoreType.DMA,
            pltpu.SemaphoreType.DMA,
            pltpu.SemaphoreType.DMA,
            pltpu.SemaphoreType.DMA,
            pltpu.SemaphoreType.DMA,
            pltpu.SemaphoreType.DMA,
            pltpu.SemaphoreType.DMA,
            pltpu.SemaphoreType.DMA,
            pltpu.SemaphoreType.DMA,
        ],
    )
    padded, mask_i = run(indices, idx_pad, table, lengths)
    return padded, mask_i.astype(jnp.bool_)
